# baseline (speedup 1.0000x reference)
import functools

import jax
import jax.numpy as jnp
from jax import lax
from jax.experimental import pallas as pl
from jax.experimental.pallas import tpu as pltpu
from jax.experimental.pallas import tpu_sc as plsc

N = 10000
E = 320000
D_IN = 128
D_HID = 256
N_CLASSES = 64
EPS = 1e-5

NP = 10240
LANES = 128
ROWS = 2560
EPAD = ROWS * LANES
PAD_IDX = N

SUB = 4
RPT_ES0 = 64
RPT_ES1 = 160 - RPT_ES0

NUM_CORES = 2
NUM_SUBCORES = 16
ROWS_PER_TILE = ROWS // NUM_SUBCORES
ROWS_PER_WORKER = ROWS // (NUM_CORES * NUM_SUBCORES)
K = 4

def _sc_mesh():
    return plsc.VectorSubcoreMesh(
        core_axis_name="c", subcore_axis_name="s",
        num_cores=NUM_CORES, num_subcores=NUM_SUBCORES)


@functools.cache
def _make_deg_kernel():
    @functools.partial(
        pl.kernel,
        out_type=jax.ShapeDtypeStruct((2 * NP,), jnp.float32),
        mesh=_sc_mesh(),
        scratch_types=[
            pltpu.VMEM((K, LANES), jnp.int32),
            pltpu.VMEM((LANES,), jnp.float32),
            pltpu.VMEM((NP // NUM_SUBCORES,), jnp.float32),
            pltpu.VMEM_SHARED((NP,), jnp.float32),
        ],
    )
    def deg_kernel(dst_rows, out, idxd, ones, zbuf, acc):
        c = lax.axis_index("c")
        s = lax.axis_index("s")
        rows_n = NP // NUM_SUBCORES
        for q in range(rows_n // 16):
            zbuf[pl.ds(q * 16, 16)] = jnp.zeros((16,), jnp.float32)
        for q in range(LANES // 16):
            ones[pl.ds(q * 16, 16)] = jnp.ones((16,), jnp.float32)
        pltpu.sync_copy(zbuf, acc.at[pl.ds(s * rows_n, rows_n)])
        plsc.subcore_barrier()

        r0 = (c * NUM_SUBCORES + s) * ROWS_PER_WORKER

        def chunk(i, carry):
            pltpu.sync_copy(dst_rows.at[pl.ds(r0 + i * K, K)], idxd)
            for j in range(K):
                pltpu.sync_copy(ones, acc.at[idxd.at[j]], add=True)
            return carry

        lax.fori_loop(0, ROWS_PER_WORKER // K, chunk, 0)
        plsc.subcore_barrier()
        pltpu.sync_copy(acc.at[pl.ds(s * rows_n, rows_n)],
                        out.at[pl.ds(c * NP + s * rows_n, rows_n)])

    return deg_kernel


@functools.cache
def _make_agg_kernel(mode):
    hd = 128
    if mode == "fs":
        NB = 16
        RPT = ROWS_PER_TILE
    else:
        NB = 8
        RPT = ROWS_PER_TILE // 2
    NBLKS = RPT // NB

    @functools.partial(
        pl.kernel,
        out_type=jax.ShapeDtypeStruct((2 * NP, hd), jnp.float32),
        mesh=_sc_mesh(),
        scratch_types=[
            pltpu.VMEM((NB, LANES), jnp.int32),
            pltpu.VMEM((NB, LANES), jnp.int32),
            pltpu.VMEM((NB, LANES), jnp.int32),
            pltpu.VMEM((NB, LANES), jnp.int32),
            pltpu.VMEM((2, LANES, hd), jnp.float32),
            pltpu.VMEM_SHARED((NP, hd), jnp.float32),
            pltpu.SemaphoreType.DMA,
            pltpu.SemaphoreType.DMA,
            pltpu.SemaphoreType.DMA,
        ],
    )
    def agg(hp, src_rows, dst_rows, out, ibs0, ibd0, ibs1, ibd1,
            gbuf, acc, sg, ss, si):
        c = lax.axis_index("c")
        s = lax.axis_index("s")
        rows_n = NP // NUM_SUBCORES
        pltpu.sync_copy(hp.at[pl.ds(2 * c * NP + s * rows_n, rows_n)],
                        acc.at[pl.ds(s * rows_n, rows_n)])
        plsc.subcore_barrier()

        src_off = c * ROWS
        if mode == "fs":
            r0 = s * RPT
            npairs = NBLKS // 2
        else:
            r0 = jnp.where(c == 0, s * RPT_ES0,
                           NUM_SUBCORES * RPT_ES0 + s * RPT_ES1)
            npairs = jnp.where(c == 0, RPT_ES0 // (2 * NB),
                               RPT_ES1 // (2 * NB))

        SR = LANES // SUB

        def fire_gathers(ib, j, q):
            for m in range(SUB):
                pltpu.async_copy(
                    hp.at[ib.at[j, pl.ds(m * SR, SR)]],
                    gbuf.at[q, pl.ds(m * SR, SR)], sg)

        def wait_gathers():
            for m in range(SUB):
                pltpu.make_async_copy(
                    hp.at[ibs0.at[0, pl.ds(0, SR)]],
                    gbuf.at[0, pl.ds(0, SR)], sg).wait()

        def wait_scatter():
            pltpu.make_async_copy(
                gbuf.at[0], acc.at[ibd0.at[0]], ss).wait()

        def wait_idx():
            pltpu.make_async_copy(
                src_rows.at[pl.ds(0, NB)], ibs0, si).wait()

        def block(i, p):
            ibs, ibd = (ibs0, ibd0) if p == 0 else (ibs1, ibd1)
            nibs, nibd = (ibs1, ibd1) if p == 0 else (ibs0, ibd0)
            base = r0 + (2 * i + p) * NB
            pltpu.async_copy(
                src_rows.at[pl.ds(src_off + base + NB, NB)], nibs, si)
            pltpu.async_copy(dst_rows.at[pl.ds(base + NB, NB)], nibd, si)
            for j in range(NB):
                q = j % 2
                if j > 0:
                    wait_scatter()
                if j < NB - 1:
                    fire_gathers(ibs, j + 1, 1 - q)
                else:
                    wait_idx()
                    wait_idx()
                    fire_gathers(nibs, 0, 1 - q)
                wait_gathers()
                pltpu.async_copy(
                    gbuf.at[q], acc.at[ibd.at[j]], ss, add=True)
            wait_scatter()

        pltpu.sync_copy(src_rows.at[pl.ds(src_off + r0, NB)], ibs0)
        pltpu.sync_copy(dst_rows.at[pl.ds(r0, NB)], ibd0)
        fire_gathers(ibs0, 0, 0)

        def pair(i, carry):
            block(i, 0)
            block(i, 1)
            return carry

        lax.fori_loop(0, npairs, pair, 0)
        wait_gathers()

        plsc.subcore_barrier()
        pltpu.sync_copy(acc.at[pl.ds(s * rows_n, rows_n)],
                        out.at[pl.ds(c * NP + s * rows_n, rows_n)])

    return agg


def _tc_first_body(x_ref, deg_ref, hp_ref, dinv_ref):
    deg = deg_ref[0:NP] + deg_ref[NP:2 * NP] + 1.0
    dinv = lax.rsqrt(deg)
    dinv_ref[...] = dinv
    hpv = x_ref[...] * dinv[0:N, None]
    z = jnp.zeros((NP - N, D_IN), jnp.float32)
    for cp in range(4):
        hp_ref[cp * NP:cp * NP + N, :] = hpv
        hp_ref[cp * NP + N:(cp + 1) * NP, :] = z


def _tc_mid1_body(a_ref, dinv_ref, x_ref, w1_ref, b_ref, g_ref, be_ref,
                  w2_ref, hp_ref):
    dinv = dinv_ref[0:N]
    aggx = a_ref[0:N, :] + a_ref[NP:NP + N, :] - x_ref[...] * dinv[:, None]
    y = jnp.dot(aggx * dinv[:, None], w1_ref[...],
                preferred_element_type=jnp.float32) + b_ref[...]
    mean = jnp.mean(y, axis=0)
    var = jnp.mean((y - mean) ** 2, axis=0)
    zz = (y - mean) * lax.rsqrt(var + EPS) * g_ref[...] + be_ref[...]
    zz = jnp.maximum(zz, 0.0)
    h = jnp.dot(zz, w2_ref[...], preferred_element_type=jnp.float32)
    hpv = h * dinv[:, None]
    z = jnp.zeros((NP - N, 128), jnp.float32)
    for cp in range(4):
        hp_ref[cp * NP:cp * NP + N, :] = hpv[:, (cp // 2) * 128:
                                             (cp // 2) * 128 + 128]
        hp_ref[cp * NP + N:(cp + 1) * NP, :] = z


def _tc_mid2_body(a_ref, dinv_ref, b_ref, g_ref, be_ref, w3_ref, hp_ref):
    dinv = dinv_ref[0:N]
    a = jnp.concatenate([a_ref[0:N, :], a_ref[NP:NP + N, :]], axis=1)
    y = a * dinv[:, None] + b_ref[...]
    mean = jnp.mean(y, axis=0)
    var = jnp.mean((y - mean) ** 2, axis=0)
    zz = (y - mean) * lax.rsqrt(var + EPS) * g_ref[...] + be_ref[...]
    zz = jnp.maximum(zz, 0.0)
    h = jnp.dot(zz, w3_ref[...], preferred_element_type=jnp.float32)
    hpv = h * dinv[:, None]
    hpw = jnp.concatenate(
        [hpv, jnp.zeros((N, 128 - N_CLASSES), jnp.float32)], axis=1)
    z = jnp.zeros((NP - N, 128), jnp.float32)
    for cp in range(4):
        hp_ref[cp * NP:cp * NP + N, :] = hpw
        hp_ref[cp * NP + N:(cp + 1) * NP, :] = z


def _tc_last_body(a_ref, dinv_ref, hp3_ref, b_ref, o_ref):
    dinv = dinv_ref[0:N]
    a = (a_ref[0:N, 0:N_CLASSES] + a_ref[NP:NP + N, 0:N_CLASSES]
         - hp3_ref[0:N, 0:N_CLASSES])
    y = a * dinv[:, None] + b_ref[...]
    m = jnp.max(y, axis=1, keepdims=True)
    lse = jnp.log(jnp.sum(jnp.exp(y - m), axis=1, keepdims=True)) + m
    o_ref[...] = y - lse


def _f32(*shape):
    return jax.ShapeDtypeStruct(shape, jnp.float32)


_tc_first = pl.pallas_call(
    _tc_first_body, out_shape=[_f32(4 * NP, D_IN), _f32(NP)])
_tc_mid1 = pl.pallas_call(
    _tc_mid1_body, out_shape=_f32(4 * NP, 128))
_tc_mid2 = pl.pallas_call(
    _tc_mid2_body, out_shape=_f32(4 * NP, 128))
_tc_last = pl.pallas_call(
    _tc_last_body, out_shape=_f32(N, N_CLASSES))


def kernel(x, edge_index, W1, b1, g1, be1, W2, b2, g2, be2, W3, b3):
    ei = edge_index.astype(jnp.int32)
    pad = jnp.full((EPAD - E,), PAD_IDX, jnp.int32)
    src = jnp.concatenate([ei[0], pad])
    dst = jnp.concatenate([ei[1], pad])
    prow = jnp.full((16 * LANES,), PAD_IDX, jnp.int32)
    dst_rows = jnp.concatenate([dst, prow]).reshape(ROWS + 16, LANES)

    row = jnp.arange(EPAD, dtype=jnp.int32) // LANES
    r_es1 = 16 * RPT_ES0
    off_es_a = ((row // RPT_ES0) % 2) * NP
    off_es_b = (2 + ((row - r_es1) // RPT_ES1) % 2) * NP
    src_es = jnp.concatenate(
        [src + off_es_a, src + off_es_b, prow]).reshape(2 * ROWS + 16, LANES)
    off_fs_a = ((row // 160) % 2) * NP
    off_fs_b = (2 + (row // 160) % 2) * NP
    src_fs = jnp.concatenate(
        [src + off_fs_a, src + off_fs_b, prow]).reshape(2 * ROWS + 16, LANES)

    degs = _make_deg_kernel()(dst_rows)
    hp1, dinv = _tc_first(x, degs)
    a1 = _make_agg_kernel("es")(hp1, src_es, dst_rows)
    hp2 = _tc_mid1(a1, dinv, x, W1, b1, g1, be1, W2)
    a2 = _make_agg_kernel("fs")(hp2, src_fs, dst_rows)
    hp3 = _tc_mid2(a2, dinv, b2, g2, be2, W3)
    a3 = _make_agg_kernel("es")(hp3, src_es, dst_rows)
    return _tc_last(a3, dinv, hp3, b3)


# speedup vs baseline: 1.0165x; 1.0165x over previous
import functools

import jax
import jax.numpy as jnp
from jax import lax
from jax.experimental import pallas as pl
from jax.experimental.pallas import tpu as pltpu
from jax.experimental.pallas import tpu_sc as plsc

N = 10000
E = 320000
D_IN = 128
D_HID = 256
N_CLASSES = 64
EPS = 1e-5

NP = 10240
LANES = 128
ROWS = 2560
EPAD = ROWS * LANES
PAD_IDX = N

SUB = 4
RPT_ES0 = 96
RPT_ES1 = 160 - RPT_ES0

NUM_CORES = 2
NUM_SUBCORES = 16
ROWS_PER_TILE = ROWS // NUM_SUBCORES
ROWS_PER_WORKER = ROWS // (NUM_CORES * NUM_SUBCORES)
K = 4

def _sc_mesh():
    return plsc.VectorSubcoreMesh(
        core_axis_name="c", subcore_axis_name="s",
        num_cores=NUM_CORES, num_subcores=NUM_SUBCORES)


@functools.cache
def _make_deg_kernel():
    @functools.partial(
        pl.kernel,
        out_type=jax.ShapeDtypeStruct((2 * NP,), jnp.float32),
        mesh=_sc_mesh(),
        scratch_types=[
            pltpu.VMEM((K, LANES), jnp.int32),
            pltpu.VMEM((LANES,), jnp.float32),
            pltpu.VMEM((NP // NUM_SUBCORES,), jnp.float32),
            pltpu.VMEM_SHARED((NP,), jnp.float32),
        ],
    )
    def deg_kernel(dst_rows, out, idxd, ones, zbuf, acc):
        c = lax.axis_index("c")
        s = lax.axis_index("s")
        rows_n = NP // NUM_SUBCORES
        for q in range(rows_n // 16):
            zbuf[pl.ds(q * 16, 16)] = jnp.zeros((16,), jnp.float32)
        for q in range(LANES // 16):
            ones[pl.ds(q * 16, 16)] = jnp.ones((16,), jnp.float32)
        pltpu.sync_copy(zbuf, acc.at[pl.ds(s * rows_n, rows_n)])
        plsc.subcore_barrier()

        r0 = (c * NUM_SUBCORES + s) * ROWS_PER_WORKER

        def chunk(i, carry):
            pltpu.sync_copy(dst_rows.at[pl.ds(r0 + i * K, K)], idxd)
            for j in range(K):
                pltpu.sync_copy(ones, acc.at[idxd.at[j]], add=True)
            return carry

        lax.fori_loop(0, ROWS_PER_WORKER // K, chunk, 0)
        plsc.subcore_barrier()
        pltpu.sync_copy(acc.at[pl.ds(s * rows_n, rows_n)],
                        out.at[pl.ds(c * NP + s * rows_n, rows_n)])

    return deg_kernel


@functools.cache
def _make_agg_kernel(mode):
    hd = 128
    if mode == "fs":
        NB = 16
        RPT = ROWS_PER_TILE
    else:
        NB = 8
        RPT = ROWS_PER_TILE // 2
    NBLKS = RPT // NB

    @functools.partial(
        pl.kernel,
        out_type=jax.ShapeDtypeStruct((2 * NP, hd), jnp.float32),
        mesh=_sc_mesh(),
        scratch_types=[
            pltpu.VMEM((NB, LANES), jnp.int32),
            pltpu.VMEM((NB, LANES), jnp.int32),
            pltpu.VMEM((NB, LANES), jnp.int32),
            pltpu.VMEM((NB, LANES), jnp.int32),
            pltpu.VMEM((2, LANES, hd), jnp.float32),
            pltpu.VMEM_SHARED((NP, hd), jnp.float32),
            pltpu.SemaphoreType.DMA,
            pltpu.SemaphoreType.DMA,
            pltpu.SemaphoreType.DMA,
        ],
    )
    def agg(hp, src_rows, dst_rows, out, ibs0, ibd0, ibs1, ibd1,
            gbuf, acc, sg, ss, si):
        c = lax.axis_index("c")
        s = lax.axis_index("s")
        rows_n = NP // NUM_SUBCORES
        pltpu.sync_copy(hp.at[pl.ds(2 * c * NP + s * rows_n, rows_n)],
                        acc.at[pl.ds(s * rows_n, rows_n)])
        plsc.subcore_barrier()

        src_off = c * ROWS
        if mode == "fs":
            r0 = s * RPT
            npairs = NBLKS // 2
        else:
            r0 = jnp.where(c == 0, s * RPT_ES0,
                           NUM_SUBCORES * RPT_ES0 + s * RPT_ES1)
            npairs = jnp.where(c == 0, RPT_ES0 // (2 * NB),
                               RPT_ES1 // (2 * NB))

        SR = LANES // SUB

        def fire_gathers(ib, j, q):
            for m in range(SUB):
                pltpu.async_copy(
                    hp.at[ib.at[j, pl.ds(m * SR, SR)]],
                    gbuf.at[q, pl.ds(m * SR, SR)], sg)

        def wait_gathers():
            for m in range(SUB):
                pltpu.make_async_copy(
                    hp.at[ibs0.at[0, pl.ds(0, SR)]],
                    gbuf.at[0, pl.ds(0, SR)], sg).wait()

        def wait_scatter():
            pltpu.make_async_copy(
                gbuf.at[0], acc.at[ibd0.at[0]], ss).wait()

        def wait_idx():
            pltpu.make_async_copy(
                src_rows.at[pl.ds(0, NB)], ibs0, si).wait()

        def block(i, p):
            ibs, ibd = (ibs0, ibd0) if p == 0 else (ibs1, ibd1)
            nibs, nibd = (ibs1, ibd1) if p == 0 else (ibs0, ibd0)
            base = r0 + (2 * i + p) * NB
            pltpu.async_copy(
                src_rows.at[pl.ds(src_off + base + NB, NB)], nibs, si)
            pltpu.async_copy(dst_rows.at[pl.ds(base + NB, NB)], nibd, si)
            for j in range(NB):
                q = j % 2
                if j > 0:
                    wait_scatter()
                if j < NB - 1:
                    fire_gathers(ibs, j + 1, 1 - q)
                else:
                    wait_idx()
                    wait_idx()
                    fire_gathers(nibs, 0, 1 - q)
                wait_gathers()
                pltpu.async_copy(
                    gbuf.at[q], acc.at[ibd.at[j]], ss, add=True)
            wait_scatter()

        pltpu.sync_copy(src_rows.at[pl.ds(src_off + r0, NB)], ibs0)
        pltpu.sync_copy(dst_rows.at[pl.ds(r0, NB)], ibd0)
        fire_gathers(ibs0, 0, 0)

        def pair(i, carry):
            block(i, 0)
            block(i, 1)
            return carry

        lax.fori_loop(0, npairs, pair, 0)
        wait_gathers()

        plsc.subcore_barrier()
        pltpu.sync_copy(acc.at[pl.ds(s * rows_n, rows_n)],
                        out.at[pl.ds(c * NP + s * rows_n, rows_n)])

    return agg


def _tc_first_body(x_ref, deg_ref, hp_ref, dinv_ref):
    deg = deg_ref[0:NP] + deg_ref[NP:2 * NP] + 1.0
    dinv = lax.rsqrt(deg)
    dinv_ref[...] = dinv
    hpv = x_ref[...] * dinv[0:N, None]
    z = jnp.zeros((NP - N, D_IN), jnp.float32)
    for cp in range(4):
        hp_ref[cp * NP:cp * NP + N, :] = hpv
        hp_ref[cp * NP + N:(cp + 1) * NP, :] = z


def _tc_mid1_body(a_ref, dinv_ref, x_ref, w1_ref, b_ref, g_ref, be_ref,
                  w2_ref, hp_ref):
    dinv = dinv_ref[0:N]
    aggx = a_ref[0:N, :] + a_ref[NP:NP + N, :] - x_ref[...] * dinv[:, None]
    y = jnp.dot(aggx * dinv[:, None], w1_ref[...],
                preferred_element_type=jnp.float32) + b_ref[...]
    mean = jnp.mean(y, axis=0)
    var = jnp.mean((y - mean) ** 2, axis=0)
    zz = (y - mean) * lax.rsqrt(var + EPS) * g_ref[...] + be_ref[...]
    zz = jnp.maximum(zz, 0.0)
    h = jnp.dot(zz, w2_ref[...], preferred_element_type=jnp.float32)
    hpv = h * dinv[:, None]
    z = jnp.zeros((NP - N, 128), jnp.float32)
    for cp in range(4):
        hp_ref[cp * NP:cp * NP + N, :] = hpv[:, (cp // 2) * 128:
                                             (cp // 2) * 128 + 128]
        hp_ref[cp * NP + N:(cp + 1) * NP, :] = z


def _tc_mid2_body(a_ref, dinv_ref, b_ref, g_ref, be_ref, w3_ref, hp_ref):
    dinv = dinv_ref[0:N]
    a = jnp.concatenate([a_ref[0:N, :], a_ref[NP:NP + N, :]], axis=1)
    y = a * dinv[:, None] + b_ref[...]
    mean = jnp.mean(y, axis=0)
    var = jnp.mean((y - mean) ** 2, axis=0)
    zz = (y - mean) * lax.rsqrt(var + EPS) * g_ref[...] + be_ref[...]
    zz = jnp.maximum(zz, 0.0)
    h = jnp.dot(zz, w3_ref[...], preferred_element_type=jnp.float32)
    hpv = h * dinv[:, None]
    hpw = jnp.concatenate(
        [hpv, jnp.zeros((N, 128 - N_CLASSES), jnp.float32)], axis=1)
    z = jnp.zeros((NP - N, 128), jnp.float32)
    for cp in range(4):
        hp_ref[cp * NP:cp * NP + N, :] = hpw
        hp_ref[cp * NP + N:(cp + 1) * NP, :] = z


def _tc_last_body(a_ref, dinv_ref, hp3_ref, b_ref, o_ref):
    dinv = dinv_ref[0:N]
    a = (a_ref[0:N, 0:N_CLASSES] + a_ref[NP:NP + N, 0:N_CLASSES]
         - hp3_ref[0:N, 0:N_CLASSES])
    y = a * dinv[:, None] + b_ref[...]
    m = jnp.max(y, axis=1, keepdims=True)
    lse = jnp.log(jnp.sum(jnp.exp(y - m), axis=1, keepdims=True)) + m
    o_ref[...] = y - lse


def _f32(*shape):
    return jax.ShapeDtypeStruct(shape, jnp.float32)


_tc_first = pl.pallas_call(
    _tc_first_body, out_shape=[_f32(4 * NP, D_IN), _f32(NP)])
_tc_mid1 = pl.pallas_call(
    _tc_mid1_body, out_shape=_f32(4 * NP, 128))
_tc_mid2 = pl.pallas_call(
    _tc_mid2_body, out_shape=_f32(4 * NP, 128))
_tc_last = pl.pallas_call(
    _tc_last_body, out_shape=_f32(N, N_CLASSES))


def kernel(x, edge_index, W1, b1, g1, be1, W2, b2, g2, be2, W3, b3):
    ei = edge_index.astype(jnp.int32)
    pad = jnp.full((EPAD - E,), PAD_IDX, jnp.int32)
    src = jnp.concatenate([ei[0], pad])
    dst = jnp.concatenate([ei[1], pad])
    prow = jnp.full((16 * LANES,), PAD_IDX, jnp.int32)
    dst_rows = jnp.concatenate([dst, prow]).reshape(ROWS + 16, LANES)

    row = jnp.arange(EPAD, dtype=jnp.int32) // LANES
    r_es1 = 16 * RPT_ES0
    off_es_a = ((row // RPT_ES0) % 2) * NP
    off_es_b = (2 + ((row - r_es1) // RPT_ES1) % 2) * NP
    src_es = jnp.concatenate(
        [src + off_es_a, src + off_es_b, prow]).reshape(2 * ROWS + 16, LANES)
    off_fs_a = ((row // 160) % 2) * NP
    off_fs_b = (2 + (row // 160) % 2) * NP
    src_fs = jnp.concatenate(
        [src + off_fs_a, src + off_fs_b, prow]).reshape(2 * ROWS + 16, LANES)

    degs = _make_deg_kernel()(dst_rows)
    hp1, dinv = _tc_first(x, degs)
    a1 = _make_agg_kernel("es")(hp1, src_es, dst_rows)
    hp2 = _tc_mid1(a1, dinv, x, W1, b1, g1, be1, W2)
    a2 = _make_agg_kernel("fs")(hp2, src_fs, dst_rows)
    hp3 = _tc_mid2(a2, dinv, b2, g2, be2, W3)
    a3 = _make_agg_kernel("es")(hp3, src_es, dst_rows)
    return _tc_last(a3, dinv, hp3, b3)


# speedup vs baseline: 1.2469x; 1.2266x over previous
import functools

import jax
import jax.numpy as jnp
from jax import lax
from jax.experimental import pallas as pl
from jax.experimental.pallas import tpu as pltpu
from jax.experimental.pallas import tpu_sc as plsc

N = 10000
E = 320000
D_IN = 128
D_HID = 256
N_CLASSES = 64
EPS = 1e-5

NP = 10240
LANES = 128
ROWS = 2560
EPAD = ROWS * LANES
PAD_IDX = N

SUB = 4
RPT_ES0 = 128
RPT_ES1 = 160 - RPT_ES0

NUM_CORES = 2
NUM_SUBCORES = 16
ROWS_PER_TILE = ROWS // NUM_SUBCORES
ROWS_PER_WORKER = ROWS // (NUM_CORES * NUM_SUBCORES)
K = 4

def _sc_mesh():
    return plsc.VectorSubcoreMesh(
        core_axis_name="c", subcore_axis_name="s",
        num_cores=NUM_CORES, num_subcores=NUM_SUBCORES)


@functools.cache
def _make_deg_kernel():
    @functools.partial(
        pl.kernel,
        out_type=jax.ShapeDtypeStruct((2 * NP,), jnp.float32),
        mesh=_sc_mesh(),
        scratch_types=[
            pltpu.VMEM((K, LANES), jnp.int32),
            pltpu.VMEM((LANES,), jnp.float32),
            pltpu.VMEM((NP // NUM_SUBCORES,), jnp.float32),
            pltpu.VMEM_SHARED((NP,), jnp.float32),
        ],
    )
    def deg_kernel(dst_rows, out, idxd, ones, zbuf, acc):
        c = lax.axis_index("c")
        s = lax.axis_index("s")
        rows_n = NP // NUM_SUBCORES
        for q in range(rows_n // 16):
            zbuf[pl.ds(q * 16, 16)] = jnp.zeros((16,), jnp.float32)
        for q in range(LANES // 16):
            ones[pl.ds(q * 16, 16)] = jnp.ones((16,), jnp.float32)
        pltpu.sync_copy(zbuf, acc.at[pl.ds(s * rows_n, rows_n)])
        plsc.subcore_barrier()

        r0 = (c * NUM_SUBCORES + s) * ROWS_PER_WORKER

        def chunk(i, carry):
            pltpu.sync_copy(dst_rows.at[pl.ds(r0 + i * K, K)], idxd)
            for j in range(K):
                pltpu.sync_copy(ones, acc.at[idxd.at[j]], add=True)
            return carry

        lax.fori_loop(0, ROWS_PER_WORKER // K, chunk, 0)
        plsc.subcore_barrier()
        pltpu.sync_copy(acc.at[pl.ds(s * rows_n, rows_n)],
                        out.at[pl.ds(c * NP + s * rows_n, rows_n)])

    return deg_kernel


@functools.cache
def _make_agg_kernel(mode):
    hd = 128
    if mode == "fs":
        NB = 16
        RPT = ROWS_PER_TILE
    else:
        NB = 8
        RPT = ROWS_PER_TILE // 2
    NBLKS = RPT // NB

    @functools.partial(
        pl.kernel,
        out_type=jax.ShapeDtypeStruct((2 * NP, hd), jnp.float32),
        mesh=_sc_mesh(),
        scratch_types=[
            pltpu.VMEM((NB, LANES), jnp.int32),
            pltpu.VMEM((NB, LANES), jnp.int32),
            pltpu.VMEM((NB, LANES), jnp.int32),
            pltpu.VMEM((NB, LANES), jnp.int32),
            pltpu.VMEM((2, LANES, hd), jnp.float32),
            pltpu.VMEM_SHARED((NP, hd), jnp.float32),
            pltpu.SemaphoreType.DMA,
            pltpu.SemaphoreType.DMA,
            pltpu.SemaphoreType.DMA,
        ],
    )
    def agg(hp, src_rows, dst_rows, out, ibs0, ibd0, ibs1, ibd1,
            gbuf, acc, sg, ss, si):
        c = lax.axis_index("c")
        s = lax.axis_index("s")
        rows_n = NP // NUM_SUBCORES
        pltpu.sync_copy(hp.at[pl.ds(2 * c * NP + s * rows_n, rows_n)],
                        acc.at[pl.ds(s * rows_n, rows_n)])
        plsc.subcore_barrier()

        src_off = c * ROWS
        if mode == "fs":
            r0 = s * RPT
            npairs = NBLKS // 2
        else:
            r0 = jnp.where(c == 0, s * RPT_ES0,
                           NUM_SUBCORES * RPT_ES0 + s * RPT_ES1)
            npairs = jnp.where(c == 0, RPT_ES0 // (2 * NB),
                               RPT_ES1 // (2 * NB))

        SR = LANES // SUB

        def fire_gathers(ib, j, q):
            for m in range(SUB):
                pltpu.async_copy(
                    hp.at[ib.at[j, pl.ds(m * SR, SR)]],
                    gbuf.at[q, pl.ds(m * SR, SR)], sg)

        def wait_gathers():
            for m in range(SUB):
                pltpu.make_async_copy(
                    hp.at[ibs0.at[0, pl.ds(0, SR)]],
                    gbuf.at[0, pl.ds(0, SR)], sg).wait()

        def wait_scatter():
            pltpu.make_async_copy(
                gbuf.at[0], acc.at[ibd0.at[0]], ss).wait()

        def wait_idx():
            pltpu.make_async_copy(
                src_rows.at[pl.ds(0, NB)], ibs0, si).wait()

        def block(i, p):
            ibs, ibd = (ibs0, ibd0) if p == 0 else (ibs1, ibd1)
            nibs, nibd = (ibs1, ibd1) if p == 0 else (ibs0, ibd0)
            base = r0 + (2 * i + p) * NB
            pltpu.async_copy(
                src_rows.at[pl.ds(src_off + base + NB, NB)], nibs, si)
            pltpu.async_copy(dst_rows.at[pl.ds(base + NB, NB)], nibd, si)
            for j in range(NB):
                q = j % 2
                if j > 0:
                    wait_scatter()
                if j < NB - 1:
                    fire_gathers(ibs, j + 1, 1 - q)
                else:
                    wait_idx()
                    wait_idx()
                    fire_gathers(nibs, 0, 1 - q)
                wait_gathers()
                pltpu.async_copy(
                    gbuf.at[q], acc.at[ibd.at[j]], ss, add=True)
            wait_scatter()

        pltpu.sync_copy(src_rows.at[pl.ds(src_off + r0, NB)], ibs0)
        pltpu.sync_copy(dst_rows.at[pl.ds(r0, NB)], ibd0)
        fire_gathers(ibs0, 0, 0)

        def pair(i, carry):
            block(i, 0)
            block(i, 1)
            return carry

        lax.fori_loop(0, npairs, pair, 0)
        wait_gathers()

        plsc.subcore_barrier()
        pltpu.sync_copy(acc.at[pl.ds(s * rows_n, rows_n)],
                        out.at[pl.ds(c * NP + s * rows_n, rows_n)])

    return agg


def _tc_first_body(x_ref, deg_ref, hp_ref, dinv_ref):
    deg = deg_ref[0:NP] + deg_ref[NP:2 * NP] + 1.0
    dinv = lax.rsqrt(deg)
    dinv_ref[...] = dinv
    hpv = x_ref[...] * dinv[0:N, None]
    z = jnp.zeros((NP - N, D_IN), jnp.float32)
    for cp in range(4):
        hp_ref[cp * NP:cp * NP + N, :] = hpv
        hp_ref[cp * NP + N:(cp + 1) * NP, :] = z


def _tc_mid1_body(a_ref, dinv_ref, x_ref, w1_ref, b_ref, g_ref, be_ref,
                  w2_ref, hp_ref):
    dinv = dinv_ref[0:N]
    aggx = a_ref[0:N, :] + a_ref[NP:NP + N, :] - x_ref[...] * dinv[:, None]
    y = jnp.dot(aggx * dinv[:, None], w1_ref[...],
                preferred_element_type=jnp.float32) + b_ref[...]
    mean = jnp.mean(y, axis=0)
    var = jnp.mean((y - mean) ** 2, axis=0)
    zz = (y - mean) * lax.rsqrt(var + EPS) * g_ref[...] + be_ref[...]
    zz = jnp.maximum(zz, 0.0)
    h = jnp.dot(zz, w2_ref[...], preferred_element_type=jnp.float32)
    hpv = h * dinv[:, None]
    z = jnp.zeros((NP - N, 128), jnp.float32)
    for cp in range(4):
        hp_ref[cp * NP:cp * NP + N, :] = hpv[:, (cp // 2) * 128:
                                             (cp // 2) * 128 + 128]
        hp_ref[cp * NP + N:(cp + 1) * NP, :] = z


def _tc_mid2_body(a_ref, dinv_ref, b_ref, g_ref, be_ref, w3_ref, hp_ref):
    dinv = dinv_ref[0:N]
    a = jnp.concatenate([a_ref[0:N, :], a_ref[NP:NP + N, :]], axis=1)
    y = a * dinv[:, None] + b_ref[...]
    mean = jnp.mean(y, axis=0)
    var = jnp.mean((y - mean) ** 2, axis=0)
    zz = (y - mean) * lax.rsqrt(var + EPS) * g_ref[...] + be_ref[...]
    zz = jnp.maximum(zz, 0.0)
    h = jnp.dot(zz, w3_ref[...], preferred_element_type=jnp.float32)
    hpv = h * dinv[:, None]
    hpw = jnp.concatenate(
        [hpv, jnp.zeros((N, 128 - N_CLASSES), jnp.float32)], axis=1)
    z = jnp.zeros((NP - N, 128), jnp.float32)
    for cp in range(4):
        hp_ref[cp * NP:cp * NP + N, :] = hpw
        hp_ref[cp * NP + N:(cp + 1) * NP, :] = z


def _tc_last_body(a_ref, dinv_ref, hp3_ref, b_ref, o_ref):
    dinv = dinv_ref[0:N]
    a = (a_ref[0:N, 0:N_CLASSES] + a_ref[NP:NP + N, 0:N_CLASSES]
         - hp3_ref[0:N, 0:N_CLASSES])
    y = a * dinv[:, None] + b_ref[...]
    m = jnp.max(y, axis=1, keepdims=True)
    lse = jnp.log(jnp.sum(jnp.exp(y - m), axis=1, keepdims=True)) + m
    o_ref[...] = y - lse


def _f32(*shape):
    return jax.ShapeDtypeStruct(shape, jnp.float32)


_tc_first = pl.pallas_call(
    _tc_first_body, out_shape=[_f32(4 * NP, D_IN), _f32(NP)])
_tc_mid1 = pl.pallas_call(
    _tc_mid1_body, out_shape=_f32(4 * NP, 128))
_tc_mid2 = pl.pallas_call(
    _tc_mid2_body, out_shape=_f32(4 * NP, 128))
_tc_last = pl.pallas_call(
    _tc_last_body, out_shape=_f32(N, N_CLASSES))


def kernel(x, edge_index, W1, b1, g1, be1, W2, b2, g2, be2, W3, b3):
    ei = edge_index.astype(jnp.int32)
    pad = jnp.full((EPAD - E,), PAD_IDX, jnp.int32)
    src = jnp.concatenate([ei[0], pad])
    dst = jnp.concatenate([ei[1], pad])
    prow = jnp.full((16 * LANES,), PAD_IDX, jnp.int32)
    dst_rows = jnp.concatenate([dst, prow]).reshape(ROWS + 16, LANES)

    row = jnp.arange(EPAD, dtype=jnp.int32) // LANES
    r_es1 = 16 * RPT_ES0
    off_es_a = ((row // RPT_ES0) % 2) * NP
    off_es_b = (2 + ((row - r_es1) // RPT_ES1) % 2) * NP
    src_es = jnp.concatenate(
        [src + off_es_a, src + off_es_b, prow]).reshape(2 * ROWS + 16, LANES)
    off_fs_a = ((row // 160) % 2) * NP
    off_fs_b = (2 + (row // 160) % 2) * NP
    src_fs = jnp.concatenate(
        [src + off_fs_a, src + off_fs_b, prow]).reshape(2 * ROWS + 16, LANES)

    degs = _make_deg_kernel()(dst_rows)
    hp1, dinv = _tc_first(x, degs)
    a1 = _make_agg_kernel("es")(hp1, src_es, dst_rows)
    hp2 = _tc_mid1(a1, dinv, x, W1, b1, g1, be1, W2)
    a2 = _make_agg_kernel("fs")(hp2, src_fs, dst_rows)
    hp3 = _tc_mid2(a2, dinv, b2, g2, be2, W3)
    a3 = _make_agg_kernel("es")(hp3, src_es, dst_rows)
    return _tc_last(a3, dinv, hp3, b3)
